# baseline (speedup 1.0000x reference)
import functools

import jax
import jax.numpy as jnp
from jax import lax
from jax.experimental import pallas as pl
from jax.experimental.pallas import tpu as pltpu
from jax.experimental.pallas import tpu_sc as plsc

_DIM = 256
_N_EMBED = 8192
_COMMITMENT = 1.0

_TM = 512


def _dist_body(x_ref, e_ref, xsq_ref, esq_ref, dist_ref, ind_ref, loss_ref,
               *, nb, inv_n):
    i = pl.program_id(0)
    x = x_ref[0] * (-2.0)
    mm = jnp.dot(x, e_ref[...], preferred_element_type=jnp.float32)
    xsq_col = xsq_ref[...].T
    dist = (xsq_col + mm) + esq_ref[...]
    dist_ref[...] = dist
    ind_ref[0, 0, :] = jnp.argmin(dist, axis=1).astype(jnp.int32)
    part = jnp.sum(jnp.min(dist, axis=1))

    @pl.when(i == 0)
    def _():
        loss_ref[0, 0] = part

    @pl.when((i > 0) & (i < nb - 1))
    def _():
        loss_ref[0, 0] += part

    @pl.when(i == nb - 1)
    def _():
        loss_ref[0, 0] = (loss_ref[0, 0] + part) * inv_n


def _dist_argmin_loss(inp, embed, xsq, esq):
    n_tok = inp.shape[0] * inp.shape[1]
    nb = n_tok // _TM
    k = inp.shape[1] // _TM
    inv_n = _COMMITMENT / float(n_tok * _DIM)
    dist, ind3, loss = pl.pallas_call(
        functools.partial(_dist_body, nb=nb, inv_n=inv_n),
        grid=(nb,),
        in_specs=[
            pl.BlockSpec((1, _TM, _DIM), lambda i: (i // k, i % k, 0)),
            pl.BlockSpec((_DIM, _N_EMBED), lambda i: (0, 0)),
            pl.BlockSpec((1, _TM), lambda i: (0, i)),
            pl.BlockSpec((1, _N_EMBED), lambda i: (0, 0)),
        ],
        out_specs=[
            pl.BlockSpec((_TM, _N_EMBED), lambda i: (i, 0)),
            pl.BlockSpec((1, 1, _TM), lambda i: (i, 0, 0)),
            pl.BlockSpec((1, 1), lambda i: (0, 0), memory_space=pltpu.SMEM),
        ],
        out_shape=[
            jax.ShapeDtypeStruct((n_tok, _N_EMBED), jnp.float32),
            jax.ShapeDtypeStruct((nb, 1, _TM), jnp.int32),
            jax.ShapeDtypeStruct((1, 1), jnp.float32),
        ],
        compiler_params=pltpu.CompilerParams(
            allow_input_fusion=[False, False, True, True]),
    )(inp, embed, xsq, esq)
    return dist, ind3.reshape(n_tok), loss.reshape(())


def _sc_gather(table, idx):
    info = plsc.get_sparse_core_info()
    nw = info.num_cores * info.num_subcores
    b = idx.shape[0]
    b_per_w = b // nw
    ch = 64
    n_ch = b_per_w // ch
    nbuf = 4
    d = table.shape[1]
    mesh = plsc.VectorSubcoreMesh(core_axis_name="c", subcore_axis_name="s")

    @functools.partial(
        pl.kernel,
        mesh=mesh,
        out_type=jax.ShapeDtypeStruct((b, d), jnp.float32),
        scratch_types=(
            [pltpu.VMEM((b_per_w,), jnp.int32)]
            + [pltpu.VMEM((ch, d), jnp.float32) for _ in range(nbuf)]
            + [pltpu.SemaphoreType.DMA for _ in range(2 * nbuf)]
        ),
    )
    def k(table_hbm, idx_hbm, out_hbm, idx_v, *rest):
        bufs = rest[:nbuf]
        gsems = rest[nbuf:2 * nbuf]
        osems = rest[2 * nbuf:]
        wid = lax.axis_index("s") * info.num_cores + lax.axis_index("c")
        base = wid * b_per_w
        pltpu.sync_copy(idx_hbm.at[pl.ds(base, b_per_w)], idx_v)
        g = [None] * n_ch
        o = [None] * n_ch
        for c in range(n_ch):
            if c >= nbuf:
                o[c - nbuf].wait()
            g[c] = pltpu.async_copy(
                table_hbm.at[idx_v.at[pl.ds(c * ch, ch)]],
                bufs[c % nbuf], gsems[c % nbuf])
            if c >= 1:
                g[c - 1].wait()
                o[c - 1] = pltpu.async_copy(
                    bufs[(c - 1) % nbuf],
                    out_hbm.at[pl.ds(base + (c - 1) * ch, ch)],
                    osems[(c - 1) % nbuf])
        g[n_ch - 1].wait()
        o[n_ch - 1] = pltpu.async_copy(
            bufs[(n_ch - 1) % nbuf],
            out_hbm.at[pl.ds(base + (n_ch - 1) * ch, ch)],
            osems[(n_ch - 1) % nbuf])
        for c in range(max(0, n_ch - nbuf), n_ch):
            o[c].wait()

    return k(table, idx)


def kernel(input, embed):
    xsq = jnp.sum(input ** 2, axis=-1).reshape(1, -1)
    esq = jnp.sum(embed ** 2, axis=0, keepdims=True)
    dist, ind, loss = _dist_argmin_loss(input, embed, xsq, esq)
    quant = _sc_gather(embed.T, ind)
    embed_ind = ind.reshape(input.shape[:-1])
    quantize = quant.reshape(input.shape)
    return quantize, embed_ind, loss, dist


# speedup vs baseline: 1.0051x; 1.0051x over previous
import functools

import jax
import jax.numpy as jnp
from jax import lax
from jax.experimental import pallas as pl
from jax.experimental.pallas import tpu as pltpu
from jax.experimental.pallas import tpu_sc as plsc

_DIM = 256
_N_EMBED = 8192
_COMMITMENT = 1.0

_TM = 512


def _dist_body(x_ref, e_ref, xsq_ref, esq_ref, dist_ref, ind_ref, loss_ref,
               *, nb, inv_n):
    i = pl.program_id(0)
    x = x_ref[0] * (-2.0)
    mm = jnp.dot(x, e_ref[...], preferred_element_type=jnp.float32)
    xsq_col = xsq_ref[...].T
    dist = (xsq_col + mm) + esq_ref[...]
    dist_ref[...] = dist
    ind_ref[0, 0, :] = jnp.argmin(dist, axis=1).astype(jnp.int32)
    part = jnp.sum(jnp.min(dist, axis=1))

    @pl.when(i == 0)
    def _():
        loss_ref[0, 0] = part

    @pl.when((i > 0) & (i < nb - 1))
    def _():
        loss_ref[0, 0] += part

    @pl.when(i == nb - 1)
    def _():
        loss_ref[0, 0] = (loss_ref[0, 0] + part) * inv_n


def _dist_argmin_loss(inp, embed, xsq, esq):
    n_tok = inp.shape[0] * inp.shape[1]
    nb = n_tok // _TM
    k = inp.shape[1] // _TM
    inv_n = _COMMITMENT / float(n_tok * _DIM)
    dist, ind3, loss = pl.pallas_call(
        functools.partial(_dist_body, nb=nb, inv_n=inv_n),
        grid=(nb,),
        in_specs=[
            pl.BlockSpec((1, _TM, _DIM), lambda i: (i // k, i % k, 0)),
            pl.BlockSpec((_DIM, _N_EMBED), lambda i: (0, 0)),
            pl.BlockSpec((1, _TM), lambda i: (0, i)),
            pl.BlockSpec((1, _N_EMBED), lambda i: (0, 0)),
        ],
        out_specs=[
            pl.BlockSpec((_TM, _N_EMBED), lambda i: (i, 0)),
            pl.BlockSpec((1, 1, _TM), lambda i: (i, 0, 0)),
            pl.BlockSpec((1, 1), lambda i: (0, 0), memory_space=pltpu.SMEM),
        ],
        out_shape=[
            jax.ShapeDtypeStruct((n_tok, _N_EMBED), jnp.float32),
            jax.ShapeDtypeStruct((nb, 1, _TM), jnp.int32),
            jax.ShapeDtypeStruct((1, 1), jnp.float32),
        ],
    )(inp, embed, xsq, esq)
    return dist, ind3.reshape(n_tok), loss.reshape(())


def _sc_gather(table, idx):
    info = plsc.get_sparse_core_info()
    nw = info.num_cores * info.num_subcores
    b = idx.shape[0]
    b_per_w = b // nw
    ch = 64
    n_ch = b_per_w // ch
    nbuf = 4
    d = table.shape[1]
    mesh = plsc.VectorSubcoreMesh(core_axis_name="c", subcore_axis_name="s")

    @functools.partial(
        pl.kernel,
        mesh=mesh,
        out_type=jax.ShapeDtypeStruct((b, d), jnp.float32),
        scratch_types=(
            [pltpu.VMEM((b_per_w,), jnp.int32)]
            + [pltpu.VMEM((ch, d), jnp.float32) for _ in range(nbuf)]
            + [pltpu.SemaphoreType.DMA for _ in range(2 * nbuf)]
        ),
    )
    def k(table_hbm, idx_hbm, out_hbm, idx_v, *rest):
        bufs = rest[:nbuf]
        gsems = rest[nbuf:2 * nbuf]
        osems = rest[2 * nbuf:]
        wid = lax.axis_index("s") * info.num_cores + lax.axis_index("c")
        base = wid * b_per_w
        pltpu.sync_copy(idx_hbm.at[pl.ds(base, b_per_w)], idx_v)
        g = [None] * n_ch
        o = [None] * n_ch
        for c in range(n_ch):
            if c >= nbuf:
                o[c - nbuf].wait()
            g[c] = pltpu.async_copy(
                table_hbm.at[idx_v.at[pl.ds(c * ch, ch)]],
                bufs[c % nbuf], gsems[c % nbuf])
            if c >= 1:
                g[c - 1].wait()
                o[c - 1] = pltpu.async_copy(
                    bufs[(c - 1) % nbuf],
                    out_hbm.at[pl.ds(base + (c - 1) * ch, ch)],
                    osems[(c - 1) % nbuf])
        g[n_ch - 1].wait()
        o[n_ch - 1] = pltpu.async_copy(
            bufs[(n_ch - 1) % nbuf],
            out_hbm.at[pl.ds(base + (n_ch - 1) * ch, ch)],
            osems[(n_ch - 1) % nbuf])
        for c in range(max(0, n_ch - nbuf), n_ch):
            o[c].wait()

    return k(table, idx)


def kernel(input, embed):
    xsq = jnp.sum(input ** 2, axis=-1).reshape(1, -1)
    esq = jnp.sum(embed ** 2, axis=0, keepdims=True)
    dist, ind, loss = _dist_argmin_loss(input, embed, xsq, esq)
    quant = _sc_gather(embed.T, ind)
    embed_ind = ind.reshape(input.shape[:-1])
    quantize = quant.reshape(input.shape)
    return quantize, embed_ind, loss, dist
